# baseline (speedup 1.0000x reference)
import functools

import jax
import jax.numpy as jnp
from jax import lax
from jax.experimental import pallas as pl
from jax.experimental.pallas import tpu as pltpu
from jax.experimental.pallas import tpu_sc as plsc

_N = 10000
_D = 128
_B = 64
_EPS = 1e-5
_LEAK = 0.0

_NC = 2
_NS = 16
_NW = _NC * _NS
_SUB = 80
_NSUB = 4
_CHUNK = _SUB * _NSUB
_LAST_SUBS = (_N - (_NW - 1) * _CHUNK) // _SUB


def _sc_segment_sums(x, batch):
    mesh = plsc.VectorSubcoreMesh(core_axis_name="c", subcore_axis_name="s")

    @functools.partial(
        pl.kernel,
        mesh=mesh,
        compiler_params=pltpu.CompilerParams(needs_layout_passes=False),
        out_type=jax.ShapeDtypeStruct((_NC, _B, _D), jnp.float32),
        scratch_types=[
            pltpu.VMEM((_NSUB, _SUB, _D), jnp.float32),
            pltpu.VMEM((_NSUB, _SUB), jnp.int32),
            pltpu.VMEM((_B, _D), jnp.float32),
            pltpu.VMEM_SHARED((_B, _D), jnp.float32),
            pltpu.SemaphoreType.DMA,
            pltpu.SemaphoreType.DMA,
            pltpu.SemaphoreType.DMA,
        ],
    )
    def seg_kernel(x_hbm, b_hbm, out_sum, xb, b_v, zv, sh_sum,
                   sem_st, sem_sc, sem_b):
        cid = lax.axis_index("c")
        sid = lax.axis_index("s")
        wid = cid * _NS + sid
        xbase = wid * _CHUNK
        last = wid == _NW - 1
        nsub = jnp.where(last, _LAST_SUBS, _NSUB)
        zeros16 = jnp.zeros((16,), jnp.float32)

        pltpu.async_copy(b_hbm.at[pl.ds(xbase, _SUB)], b_v.at[0], sem_b)
        pltpu.async_copy(x_hbm.at[pl.ds(xbase, _SUB)], xb.at[0], sem_st)

        @pl.when(jnp.logical_not(last))
        def _():
            for g in range(1, _NSUB):
                pltpu.async_copy(b_hbm.at[pl.ds(xbase + g * _SUB, _SUB)],
                                 b_v.at[g], sem_b)
                pltpu.async_copy(x_hbm.at[pl.ds(xbase + g * _SUB, _SUB)],
                                 xb.at[g], sem_st)

        @pl.when(sid == 0)
        def _():
            def _zrow(r, c):
                for j in range(_D // 16):
                    zv[r, pl.ds(j * 16, 16)] = zeros16
                return c
            lax.fori_loop(0, _B, _zrow, 0)
            pltpu.sync_copy(zv, sh_sum)

        plsc.subcore_barrier()

        def _run(g):
            pltpu.make_async_copy(
                x_hbm.at[pl.ds(xbase + g * _SUB, _SUB)], xb.at[g],
                sem_st).wait()
            pltpu.make_async_copy(
                b_hbm.at[pl.ds(xbase + g * _SUB, _SUB)], b_v.at[g],
                sem_b).wait()
            pltpu.async_copy(xb.at[g], sh_sum.at[b_v.at[g]], sem_sc,
                             add=True)

        _run(0)

        @pl.when(jnp.logical_not(last))
        def _():
            for g in range(1, _NSUB):
                _run(g)

        def _drain(g):
            pltpu.make_async_copy(xb.at[g], sh_sum.at[b_v.at[g]],
                                  sem_sc).wait()

        _drain(0)

        @pl.when(jnp.logical_not(last))
        def _():
            for g in range(1, _NSUB):
                _drain(g)

        plsc.subcore_barrier()

        @pl.when(sid == 0)
        def _():
            pltpu.sync_copy(sh_sum, out_sum.at[cid])

    return seg_kernel(x, batch)


def _tc_mlp(psum, batch, u, g1, be1, W1, c1, g2, be2, W2, c2,
            g3, be3, W3, c3):
    def body(ps, b_r, u_r, g1_r, be1_r, W1_r, c1_r, g2_r, be2_r, W2_r, c2_r,
             g3_r, be3_r, W3_r, c3_r, out):
        s = ps[0] + ps[1]
        seg_ids = lax.broadcasted_iota(jnp.int32, (_B, 1), 0)
        b_row = b_r[...][None, :]
        onehot = (b_row == seg_ids).astype(jnp.float32)
        ones_col = jnp.ones((_N, 1), jnp.float32)
        cnt = jnp.dot(onehot, ones_col,
                      preferred_element_type=jnp.float32)
        pooled = s / jnp.clip(cnt, 1.0)
        h = jnp.concatenate([u_r[...], pooled], axis=1)

        def bn(h, g_v, b_v):
            mu = jnp.mean(h, axis=0, keepdims=True)
            var = jnp.mean((h - mu) * (h - mu), axis=0, keepdims=True)
            return (g_v[...][None, :] * (h - mu) * lax.rsqrt(var + _EPS)
                    + b_v[...][None, :])

        def lrelu(h):
            return jnp.where(h >= 0, h, _LEAK * h)

        h = bn(h, g1_r, be1_r)
        h = lrelu(jnp.dot(h, W1_r[...], preferred_element_type=jnp.float32)
                  + c1_r[...][None, :])
        h = bn(h, g2_r, be2_r)
        h = lrelu(jnp.dot(h, W2_r[...], preferred_element_type=jnp.float32)
                  + c2_r[...][None, :])
        h = bn(h, g3_r, be3_r)
        out[...] = (jnp.dot(h, W3_r[...], preferred_element_type=jnp.float32)
                    + c3_r[...][None, :])

    return pl.pallas_call(
        body,
        out_shape=jax.ShapeDtypeStruct((_B, W3.shape[1]), jnp.float32),
    )(psum, batch, u, g1, be1, W1, c1, g2, be2, W2, c2, g3, be3, W3, c3)


def kernel(x, edge_index, edge_attr, u, batch,
           g1, be1, W1, c1, g2, be2, W2, c2, g3, be3, W3, c3):
    del edge_index, edge_attr
    psum = _sc_segment_sums(x, batch)
    return _tc_mlp(psum, batch, u, g1, be1, W1, c1,
                   g2, be2, W2, c2, g3, be3, W3, c3)
